# Initial kernel scaffold; baseline (speedup 1.0000x reference)
import jax
import jax.numpy as jnp
from jax.experimental import pallas as pl


def kernel(input):
    raise NotImplementedError("write your pallas kernel here")


# speedup vs baseline: 5.2058x; 5.2058x over previous
import functools

import jax
import jax.numpy as jnp
from jax.experimental import pallas as pl

_K = 10
_N = 1024
_ROWS_PER_BLOCK = 256


def _topk_sum_block(x_ref, o_ref):
    x = x_ref[...]
    neg = jnp.float32(-jnp.inf)

    def body(_, carry):
        s, cnt, rem = carry
        m = jnp.max(rem, axis=1, keepdims=True)
        eq = rem == m
        c = jnp.sum(eq.astype(jnp.float32), axis=1, keepdims=True)
        take = jnp.minimum(c, jnp.float32(_K) - cnt)
        s = s + take * m
        cnt = cnt + take
        rem = jnp.where(eq, neg, rem)
        return s, cnt, rem

    r = x.shape[0]
    s0 = jnp.zeros((r, 1), jnp.float32)
    s, _, _ = jax.lax.fori_loop(0, _K, body, (s0, s0, x))
    o_ref[...] = s[:, 0] * jnp.float32(1.0 / _K)


@functools.partial(jax.jit, static_argnames=("interpret",))
def kernel(input, interpret=False):
    b, c, h, w = input.shape
    n = h * w
    rows = b * c
    flat = input.reshape(rows, n)
    rpb = min(_ROWS_PER_BLOCK, rows)
    grid = rows // rpb
    out = pl.pallas_call(
        _topk_sum_block,
        grid=(grid,),
        in_specs=[pl.BlockSpec((rpb, n), lambda i: (i, 0))],
        out_specs=pl.BlockSpec((rpb,), lambda i: (i,)),
        out_shape=jax.ShapeDtypeStruct((rows,), jnp.float32),
        interpret=interpret,
    )(flat)
    return out.reshape(b, c)


# speedup vs baseline: 7.3782x; 1.4173x over previous
import functools

import jax
import jax.numpy as jnp
from jax import lax
from jax.experimental import pallas as pl
from jax.experimental.pallas import tpu as pltpu
from jax.experimental.pallas import tpu_sc as plsc

_K = 10
_N = 1024
_ROWS = 24576
_NC, _NS, _L = 2, 16, 16
_NW = _NC * _NS
_RPW = _ROWS // _NW
_CH = 16
_NCHUNK = _RPW // _CH


def _merge_kv(ak, av, bk, bv):
    rk = lax.rev(bk, (0,))
    rv = lax.rev(bv, (0,))
    take = ak >= rk
    return jnp.where(take, ak, rk), jnp.where(take, av, rv)


def _merge_k(a, b):
    return jnp.maximum(a, lax.rev(b, (0,)))


def _sort_k(x):
    s, _ = plsc.sort_key_val(x, lax.iota(jnp.int32, _L))
    return s


def _row_topk_sum(buf, roff):
    iota = lax.iota(jnp.int32, _L)

    pairs = []
    for j in range(8):
        g = buf[pl.ds(roff + j * 128, _L)]
        for s in range(1, 8):
            g = jnp.maximum(g, buf[pl.ds(roff + j * 128 + s * 16, _L)])
        sk, sv = plsc.sort_key_val(g, iota + j * 128)
        pairs.append((sk, sv))

    l1 = []
    for i in (0, 2, 4, 6):
        mk, mv = _merge_kv(*pairs[i], *pairs[i + 1])
        l1.append(plsc.sort_key_val(mk, mv))
    mk0, mv0 = _merge_kv(*l1[0], *l1[1])
    mk1, mv1 = _merge_kv(*l1[2], *l1[3])
    p0 = plsc.sort_key_val(mk0, mv0)
    p1 = plsc.sort_key_val(mk1, mv1)
    _, bases = _merge_kv(*p0, *p1)

    idx0 = bases + roff
    cands = [plsc.load_gather(buf, [idx0 + s * 16]) for s in range(8)]

    ss = [_sort_k(c) for c in cands]
    m1 = [_sort_k(_merge_k(ss[i], ss[i + 1])) for i in (0, 2, 4, 6)]
    m2 = [_sort_k(_merge_k(m1[0], m1[1])), _sort_k(_merge_k(m1[2], m1[3]))]
    top16 = _sort_k(_merge_k(m2[0], m2[1]))
    return jnp.sum(jnp.where(iota >= _L - _K, top16, jnp.float32(0.0)))


def _sc_body(x_hbm, out_hbm, buf, out_v, sem):
    wid = lax.axis_index("s") * _NC + lax.axis_index("c")
    wbase = wid * _RPW
    iota = lax.iota(jnp.int32, _L)

    def chunk_body(ci, _):
        rowbase = wbase + ci * _CH
        pltpu.async_copy(
            x_hbm.at[pl.ds(rowbase * _N, _CH * _N)], buf, sem
        ).wait()

        def row_body(r, acc):
            s = _row_topk_sum(buf, r * _N)
            return jnp.where(iota == r, s * jnp.float32(1.0 / _K), acc)

        acc = lax.fori_loop(0, _CH, row_body, jnp.zeros((_L,), jnp.float32))
        out_v[pl.ds(ci * _CH, _CH)] = acc
        return 0

    lax.fori_loop(0, _NCHUNK, chunk_body, 0)
    pltpu.sync_copy(out_v, out_hbm.at[pl.ds(wbase, _RPW)])


@jax.jit
def kernel(input):
    b, c, h, w = input.shape
    flat = input.reshape(b * c * h * w)
    mesh = plsc.VectorSubcoreMesh(
        core_axis_name="c", subcore_axis_name="s",
        num_cores=_NC, num_subcores=_NS,
    )
    out = pl.kernel(
        _sc_body,
        out_type=jax.ShapeDtypeStruct((_ROWS,), jnp.float32),
        mesh=mesh,
        compiler_params=pltpu.CompilerParams(needs_layout_passes=False),
        scratch_types=[
            pltpu.VMEM((_CH * _N,), jnp.float32),
            pltpu.VMEM((_RPW,), jnp.float32),
            pltpu.SemaphoreType.DMA,
        ],
    )(flat)
    return out.reshape(b, c)
